# baseline (speedup 1.0000x reference)
import functools

import jax
import jax.numpy as jnp
from jax import lax
from jax.experimental import pallas as pl
from jax.experimental.pallas import tpu as pltpu
from jax.experimental.pallas import tpu_sc as plsc

N = 10000
D = 128
H = 256
K = 16
E = 320000

NC = 2
NS = 16
NW = NC * NS
CHUNK = 128
TOTCH = 2560
CHD = TOTCH // NW
CH0 = 56
CH1 = (TOTCH // NS) - CH0
CHMAX = max(CH0, CH1)
TOTCHP = TOTCH + CHMAX
EPAD = TOTCHP * CHUNK
NP = 10112
RPS = NP // NS
NPD = 10240
RPSD = NPD // NS

ROWBLK = 1000
GB = N // ROWBLK

_mesh = plsc.VectorSubcoreMesh(
    core_axis_name="c", subcore_axis_name="s", num_cores=NC, num_subcores=NS
)


@functools.partial(
    pl.kernel,
    out_type=jax.ShapeDtypeStruct((NC, NPD), jnp.float32),
    mesh=_mesh,
    scratch_types=[
        pltpu.VMEM((CHD, CHUNK), jnp.int32),
        pltpu.VMEM((CHUNK,), jnp.float32),
        pltpu.VMEM_SHARED((NPD,), jnp.float32),
    ],
)
def _sc_degree(dstw, zer, out, didx, ones_v, dacc):
    cid = lax.axis_index("c")
    sid = lax.axis_index("s")
    wid = cid * NS + sid
    pltpu.sync_copy(zer.at[pl.ds(sid * RPSD, RPSD)], dacc.at[pl.ds(sid * RPSD, RPSD)])
    pltpu.sync_copy(dstw.at[pl.ds(wid * CHD, CHD)], didx)

    def initones(i, c):
        ones_v[pl.ds(i * 16, 16)] = jnp.ones((16,), jnp.float32)
        return c

    lax.fori_loop(0, CHUNK // 16, initones, 0)
    plsc.subcore_barrier()

    def body(j, c):
        pltpu.sync_copy(ones_v, dacc.at[didx.at[j]], add=True)
        return c

    lax.fori_loop(0, CHD, body, 0)
    plsc.subcore_barrier()
    pltpu.sync_copy(dacc.at[pl.ds(sid * RPSD, RPSD)], out.at[cid, pl.ds(sid * RPSD, RPSD)])


@functools.partial(
    pl.kernel,
    out_type=jax.ShapeDtypeStruct((NC, NP, D), jnp.float32),
    mesh=_mesh,
    scratch_types=[
        pltpu.VMEM((CHMAX, CHUNK), jnp.int32),
        pltpu.VMEM((CHMAX, CHUNK), jnp.int32),
        pltpu.VMEM((CHUNK, D), jnp.float32),
        pltpu.VMEM_SHARED((NP, D), jnp.float32),
        pltpu.SemaphoreType.DMA,
    ],
)
def _sc_spmm(hp, srcw, dstw, zer, out, sidx, didx, rows, acc, sem):
    cid = lax.axis_index("c")
    sid = lax.axis_index("s")
    nch = jnp.where(cid == 0, CH0, CH1)
    base = jnp.where(cid == 0, sid * CH0, NS * CH0 + sid * CH1)
    pltpu.sync_copy(zer.at[pl.ds(sid * RPS, RPS)], acc.at[pl.ds(sid * RPS, RPS)])
    pltpu.sync_copy(srcw.at[pl.ds(base, CHMAX)], sidx)
    pltpu.sync_copy(dstw.at[pl.ds(base, CHMAX)], didx)
    plsc.subcore_barrier()

    def body(j, c):
        pltpu.async_copy(hp.at[sidx.at[j]], rows, sem).wait()
        pltpu.sync_copy(rows, acc.at[didx.at[j]], add=True)
        return c

    lax.fori_loop(0, nch, body, 0)
    plsc.subcore_barrier()
    pltpu.sync_copy(acc.at[pl.ds(sid * RPS, RPS)], out.at[cid, pl.ds(sid * RPS, RPS)])


def _dinv_body(d0, d1, o):
    o[...] = lax.rsqrt(d0[...] + d1[...] + 1.0)


def _dinv(d0, d1):
    return pl.pallas_call(
        _dinv_body,
        out_shape=jax.ShapeDtypeStruct(d0.shape, jnp.float32),
    )(d0, d1)


def _mm_scale_body(h, w, dv, o):
    o[...] = (
        jnp.dot(h[...], w[...], preferred_element_type=jnp.float32,
                precision=lax.Precision.HIGHEST)
        * dv[...]
    )


def _mm_scale(h, w, dinv_col):
    return pl.pallas_call(
        _mm_scale_body,
        grid=(GB,),
        in_specs=[
            pl.BlockSpec((ROWBLK, D), lambda i: (i, 0)),
            pl.BlockSpec((D, D), lambda i: (0, 0)),
            pl.BlockSpec((ROWBLK, 1), lambda i: (i, 0)),
        ],
        out_specs=pl.BlockSpec((ROWBLK, D), lambda i: (i, 0)),
        out_shape=jax.ShapeDtypeStruct((N, D), jnp.float32),
    )(h, w, dinv_col)


def _finish_body(a0, a1, hp, dv, b, g, be, res, o):
    t = (a0[...] + a1[...] + hp[...]) * dv[...] + b[...]
    mu = jnp.mean(t, axis=-1, keepdims=True)
    xc = t - mu
    var = jnp.mean(xc * xc, axis=-1, keepdims=True)
    y = xc * lax.rsqrt(var + 1e-5) * g[...] + be[...]
    e = jnp.where(y > 0, y, jnp.exp(y) - 1.0)
    o[...] = e + res[...]


def _finish(a0, a1, hp, dinv_col, b, g, be, res):
    blk = lambda i: (i, 0)
    return pl.pallas_call(
        _finish_body,
        grid=(GB,),
        in_specs=[
            pl.BlockSpec((ROWBLK, D), blk),
            pl.BlockSpec((ROWBLK, D), blk),
            pl.BlockSpec((ROWBLK, D), blk),
            pl.BlockSpec((ROWBLK, 1), blk),
            pl.BlockSpec((1, D), lambda i: (0, 0)),
            pl.BlockSpec((1, D), lambda i: (0, 0)),
            pl.BlockSpec((1, D), lambda i: (0, 0)),
            pl.BlockSpec((ROWBLK, D), blk),
        ],
        out_specs=pl.BlockSpec((ROWBLK, D), blk),
        out_shape=jax.ShapeDtypeStruct((N, D), jnp.float32),
    )(a0, a1, hp, dinv_col, b, g, be, res)


def _head_body(h, w1, b1, g, be, w2, b2, o):
    m = (
        jnp.dot(h[...], w1[...], preferred_element_type=jnp.float32,
                precision=lax.Precision.HIGHEST)
        + b1[...]
    )
    mu = jnp.mean(m, axis=-1, keepdims=True)
    xc = m - mu
    var = jnp.mean(xc * xc, axis=-1, keepdims=True)
    y = xc * lax.rsqrt(var + 1e-5) * g[...] + be[...]
    e = jnp.where(y > 0, y, jnp.exp(y) - 1.0)
    lg = (
        jnp.dot(e, w2[...], preferred_element_type=jnp.float32,
                precision=lax.Precision.HIGHEST)
        + b2[...]
    )
    z = lg - jnp.max(lg, axis=-1, keepdims=True)
    ez = jnp.exp(z)
    o[...] = ez / jnp.sum(ez, axis=-1, keepdims=True)


def _head(h, w1, b1, g, be, w2, b2):
    return pl.pallas_call(
        _head_body,
        grid=(GB,),
        in_specs=[
            pl.BlockSpec((ROWBLK, D), lambda i: (i, 0)),
            pl.BlockSpec((D, H), lambda i: (0, 0)),
            pl.BlockSpec((1, H), lambda i: (0, 0)),
            pl.BlockSpec((1, H), lambda i: (0, 0)),
            pl.BlockSpec((1, H), lambda i: (0, 0)),
            pl.BlockSpec((H, K), lambda i: (0, 0)),
            pl.BlockSpec((1, K), lambda i: (0, 0)),
        ],
        out_specs=pl.BlockSpec((ROWBLK, K), lambda i: (i, 0)),
        out_shape=jax.ShapeDtypeStruct((N, K), jnp.float32),
    )(h, w1, b1, g, be, w2, b2)


def kernel(x, edge_index, W1, b1, g1, be1, W2, b2, g2, be2, Wm1, bm1, gm, bem, Wm2, bm2):
    src = edge_index[0]
    dst = edge_index[1]
    pad = EPAD - E
    srcw = jnp.concatenate([src, jnp.zeros((pad,), jnp.int32)]).reshape(TOTCHP, CHUNK)
    dstw = jnp.concatenate([dst, jnp.full((pad,), N, jnp.int32)]).reshape(TOTCHP, CHUNK)
    zer2 = jnp.zeros((NP, D), jnp.float32)
    zer1 = jnp.zeros((NPD,), jnp.float32)

    degp = _sc_degree(dstw, zer1)
    dinv2d = _dinv(degp[0].reshape(NPD // 128, 128), degp[1].reshape(NPD // 128, 128))
    dinv_col = dinv2d.reshape(NPD)[:N].reshape(N, 1)

    b1r, g1r, be1r = b1.reshape(1, D), g1.reshape(1, D), be1.reshape(1, D)
    b2r, g2r, be2r = b2.reshape(1, D), g2.reshape(1, D), be2.reshape(1, D)

    h = x
    for (W, bb, gg, bebe) in ((W1, b1r, g1r, be1r), (W2, b2r, g2r, be2r)):
        hp = _mm_scale(h, W, dinv_col)
        agg = _sc_spmm(hp, srcw, dstw, zer2)
        h = _finish(agg[0, :N], agg[1, :N], hp, dinv_col, bb, gg, bebe, h)

    return _head(
        h, Wm1, bm1.reshape(1, H), gm.reshape(1, H), bem.reshape(1, H),
        Wm2, bm2.reshape(1, K),
    )


# speedup vs baseline: 1.1458x; 1.1458x over previous
import functools

import jax
import jax.numpy as jnp
from jax import lax
from jax.experimental import pallas as pl
from jax.experimental.pallas import tpu as pltpu
from jax.experimental.pallas import tpu_sc as plsc

N = 10000
D = 128
H = 256
K = 16
E = 320000

NC = 2
NS = 16
NW = NC * NS
CHUNK = 128
TOTCH = 2560
CHD = TOTCH // NW
CH0 = 104
CH1 = (TOTCH // NS) - CH0
CHMAX = max(CH0, CH1)
TOTCHP = TOTCH + CHMAX
EPAD = TOTCHP * CHUNK
NP = 10112
RPS = NP // NS
NPD = 10240
RPSD = NPD // NS

ROWBLK = 1000
GB = N // ROWBLK

_mesh = plsc.VectorSubcoreMesh(
    core_axis_name="c", subcore_axis_name="s", num_cores=NC, num_subcores=NS
)


@functools.partial(
    pl.kernel,
    out_type=jax.ShapeDtypeStruct((NC, NPD), jnp.float32),
    mesh=_mesh,
    scratch_types=[
        pltpu.VMEM((CHD, CHUNK), jnp.int32),
        pltpu.VMEM((CHUNK,), jnp.float32),
        pltpu.VMEM_SHARED((NPD,), jnp.float32),
    ],
)
def _sc_degree(dstw, zer, out, didx, ones_v, dacc):
    cid = lax.axis_index("c")
    sid = lax.axis_index("s")
    wid = cid * NS + sid
    pltpu.sync_copy(zer.at[pl.ds(sid * RPSD, RPSD)], dacc.at[pl.ds(sid * RPSD, RPSD)])
    pltpu.sync_copy(dstw.at[pl.ds(wid * CHD, CHD)], didx)

    def initones(i, c):
        ones_v[pl.ds(i * 16, 16)] = jnp.ones((16,), jnp.float32)
        return c

    lax.fori_loop(0, CHUNK // 16, initones, 0)
    plsc.subcore_barrier()

    def body(j, c):
        pltpu.sync_copy(ones_v, dacc.at[didx.at[j]], add=True)
        return c

    lax.fori_loop(0, CHD, body, 0)
    plsc.subcore_barrier()
    pltpu.sync_copy(dacc.at[pl.ds(sid * RPSD, RPSD)], out.at[cid, pl.ds(sid * RPSD, RPSD)])


@functools.partial(
    pl.kernel,
    out_type=jax.ShapeDtypeStruct((NC, NP, D), jnp.float32),
    mesh=_mesh,
    scratch_types=[
        pltpu.VMEM((CHMAX, CHUNK), jnp.int32),
        pltpu.VMEM((CHMAX, CHUNK), jnp.int32),
        pltpu.VMEM((CHUNK, D), jnp.float32),
        pltpu.VMEM_SHARED((NP, D), jnp.float32),
        pltpu.SemaphoreType.DMA,
    ],
)
def _sc_spmm(hp, srcw, dstw, zer, out, sidx, didx, rows, acc, sem):
    cid = lax.axis_index("c")
    sid = lax.axis_index("s")
    nch = jnp.where(cid == 0, CH0, CH1)
    base = jnp.where(cid == 0, sid * CH0, NS * CH0 + sid * CH1)
    pltpu.sync_copy(zer.at[pl.ds(sid * RPS, RPS)], acc.at[pl.ds(sid * RPS, RPS)])
    pltpu.sync_copy(srcw.at[pl.ds(base, CHMAX)], sidx)
    pltpu.sync_copy(dstw.at[pl.ds(base, CHMAX)], didx)
    plsc.subcore_barrier()

    def body(j, c):
        pltpu.async_copy(hp.at[sidx.at[j]], rows, sem).wait()
        pltpu.sync_copy(rows, acc.at[didx.at[j]], add=True)
        return c

    lax.fori_loop(0, nch, body, 0)
    plsc.subcore_barrier()
    pltpu.sync_copy(acc.at[pl.ds(sid * RPS, RPS)], out.at[cid, pl.ds(sid * RPS, RPS)])


def _dinv_body(d0, d1, o):
    o[...] = lax.rsqrt(d0[...] + d1[...] + 1.0)


def _dinv(d0, d1):
    return pl.pallas_call(
        _dinv_body,
        out_shape=jax.ShapeDtypeStruct(d0.shape, jnp.float32),
    )(d0, d1)


def _mm_scale_body(h, w, dv, o):
    o[...] = (
        jnp.dot(h[...], w[...], preferred_element_type=jnp.float32,
                precision=lax.Precision.HIGHEST)
        * dv[...]
    )


def _mm_scale(h, w, dinv_col):
    return pl.pallas_call(
        _mm_scale_body,
        grid=(GB,),
        in_specs=[
            pl.BlockSpec((ROWBLK, D), lambda i: (i, 0)),
            pl.BlockSpec((D, D), lambda i: (0, 0)),
            pl.BlockSpec((ROWBLK, 1), lambda i: (i, 0)),
        ],
        out_specs=pl.BlockSpec((ROWBLK, D), lambda i: (i, 0)),
        out_shape=jax.ShapeDtypeStruct((N, D), jnp.float32),
    )(h, w, dinv_col)


def _finish_body(a0, a1, hp, dv, b, g, be, res, o):
    t = (a0[...] + a1[...] + hp[...]) * dv[...] + b[...]
    mu = jnp.mean(t, axis=-1, keepdims=True)
    xc = t - mu
    var = jnp.mean(xc * xc, axis=-1, keepdims=True)
    y = xc * lax.rsqrt(var + 1e-5) * g[...] + be[...]
    e = jnp.where(y > 0, y, jnp.exp(y) - 1.0)
    o[...] = e + res[...]


def _finish(a0, a1, hp, dinv_col, b, g, be, res):
    blk = lambda i: (i, 0)
    return pl.pallas_call(
        _finish_body,
        grid=(GB,),
        in_specs=[
            pl.BlockSpec((ROWBLK, D), blk),
            pl.BlockSpec((ROWBLK, D), blk),
            pl.BlockSpec((ROWBLK, D), blk),
            pl.BlockSpec((ROWBLK, 1), blk),
            pl.BlockSpec((1, D), lambda i: (0, 0)),
            pl.BlockSpec((1, D), lambda i: (0, 0)),
            pl.BlockSpec((1, D), lambda i: (0, 0)),
            pl.BlockSpec((ROWBLK, D), blk),
        ],
        out_specs=pl.BlockSpec((ROWBLK, D), blk),
        out_shape=jax.ShapeDtypeStruct((N, D), jnp.float32),
    )(a0, a1, hp, dinv_col, b, g, be, res)


def _head_body(h, w1, b1, g, be, w2, b2, o):
    m = (
        jnp.dot(h[...], w1[...], preferred_element_type=jnp.float32,
                precision=lax.Precision.HIGHEST)
        + b1[...]
    )
    mu = jnp.mean(m, axis=-1, keepdims=True)
    xc = m - mu
    var = jnp.mean(xc * xc, axis=-1, keepdims=True)
    y = xc * lax.rsqrt(var + 1e-5) * g[...] + be[...]
    e = jnp.where(y > 0, y, jnp.exp(y) - 1.0)
    lg = (
        jnp.dot(e, w2[...], preferred_element_type=jnp.float32,
                precision=lax.Precision.HIGHEST)
        + b2[...]
    )
    z = lg - jnp.max(lg, axis=-1, keepdims=True)
    ez = jnp.exp(z)
    o[...] = ez / jnp.sum(ez, axis=-1, keepdims=True)


def _head(h, w1, b1, g, be, w2, b2):
    return pl.pallas_call(
        _head_body,
        grid=(GB,),
        in_specs=[
            pl.BlockSpec((ROWBLK, D), lambda i: (i, 0)),
            pl.BlockSpec((D, H), lambda i: (0, 0)),
            pl.BlockSpec((1, H), lambda i: (0, 0)),
            pl.BlockSpec((1, H), lambda i: (0, 0)),
            pl.BlockSpec((1, H), lambda i: (0, 0)),
            pl.BlockSpec((H, K), lambda i: (0, 0)),
            pl.BlockSpec((1, K), lambda i: (0, 0)),
        ],
        out_specs=pl.BlockSpec((ROWBLK, K), lambda i: (i, 0)),
        out_shape=jax.ShapeDtypeStruct((N, K), jnp.float32),
    )(h, w1, b1, g, be, w2, b2)


def kernel(x, edge_index, W1, b1, g1, be1, W2, b2, g2, be2, Wm1, bm1, gm, bem, Wm2, bm2):
    src = edge_index[0]
    dst = edge_index[1]
    pad = EPAD - E
    srcw = jnp.concatenate([src, jnp.zeros((pad,), jnp.int32)]).reshape(TOTCHP, CHUNK)
    dstw = jnp.concatenate([dst, jnp.full((pad,), N, jnp.int32)]).reshape(TOTCHP, CHUNK)
    zer2 = jnp.zeros((NP, D), jnp.float32)
    zer1 = jnp.zeros((NPD,), jnp.float32)

    degp = _sc_degree(dstw, zer1)
    dinv2d = _dinv(degp[0].reshape(NPD // 128, 128), degp[1].reshape(NPD // 128, 128))
    dinv_col = dinv2d.reshape(NPD)[:N].reshape(N, 1)

    b1r, g1r, be1r = b1.reshape(1, D), g1.reshape(1, D), be1.reshape(1, D)
    b2r, g2r, be2r = b2.reshape(1, D), g2.reshape(1, D), be2.reshape(1, D)

    h = x
    for (W, bb, gg, bebe) in ((W1, b1r, g1r, be1r), (W2, b2r, g2r, be2r)):
        hp = _mm_scale(h, W, dinv_col)
        agg = _sc_spmm(hp, srcw, dstw, zer2)
        h = _finish(agg[0, :N], agg[1, :N], hp, dinv_col, bb, gg, bebe, h)

    return _head(
        h, Wm1, bm1.reshape(1, H), gm.reshape(1, H), bem.reshape(1, H),
        Wm2, bm2.reshape(1, K),
    )


# speedup vs baseline: 2.7822x; 2.4281x over previous
import functools

import jax
import jax.numpy as jnp
from jax import lax
from jax.experimental import pallas as pl
from jax.experimental.pallas import tpu as pltpu
from jax.experimental.pallas import tpu_sc as plsc

N = 10000
D = 128
H = 256
K = 16
E = 320000

NC = 2
NS = 16
NW = NC * NS
CHUNK = 128
TOTCH = 2560
CHD = TOTCH // NW
EPAD = TOTCH * CHUNK
NDUMP = 240
NP = 10240
RPS = NP // NS
NPD = 10240
RPSD = NPD // NS

ROWBLK = 1000
GB = N // ROWBLK

_mesh = plsc.VectorSubcoreMesh(
    core_axis_name="c", subcore_axis_name="s", num_cores=NC, num_subcores=NS
)


@functools.partial(
    pl.kernel,
    out_type=jax.ShapeDtypeStruct((NC, NPD), jnp.float32),
    mesh=_mesh,
    scratch_types=[
        pltpu.VMEM((CHD, CHUNK), jnp.int32),
        pltpu.VMEM((CHUNK,), jnp.float32),
        pltpu.VMEM_SHARED((NPD,), jnp.float32),
    ],
)
def _sc_degree(dstw, zer, out, didx, ones_v, dacc):
    cid = lax.axis_index("c")
    sid = lax.axis_index("s")
    wid = cid * NS + sid
    pltpu.sync_copy(zer.at[pl.ds(sid * RPSD, RPSD)], dacc.at[pl.ds(sid * RPSD, RPSD)])
    pltpu.sync_copy(dstw.at[pl.ds(wid * CHD, CHD)], didx)

    def initones(i, c):
        ones_v[pl.ds(i * 16, 16)] = jnp.ones((16,), jnp.float32)
        return c

    lax.fori_loop(0, CHUNK // 16, initones, 0)
    plsc.subcore_barrier()

    def body(j, c):
        pltpu.sync_copy(ones_v, dacc.at[didx.at[j]], add=True)
        return c

    lax.fori_loop(0, CHD, body, 0)
    plsc.subcore_barrier()
    pltpu.sync_copy(dacc.at[pl.ds(sid * RPSD, RPSD)], out.at[cid, pl.ds(sid * RPSD, RPSD)])


@functools.partial(
    pl.kernel,
    out_type=jax.ShapeDtypeStruct((NC, NP, D), jnp.float32),
    mesh=_mesh,
    scratch_types=[
        pltpu.VMEM((CHD, CHUNK), jnp.int32),
        pltpu.VMEM((CHD, CHUNK), jnp.int32),
        pltpu.VMEM((CHUNK, D), jnp.float32),
        pltpu.VMEM_SHARED((NP, D), jnp.float32),
        pltpu.SemaphoreType.DMA,
    ],
)
def _sc_spmm(hp, srcw, dstw, zer, out, sidx, didx, rows, acc, sem):
    cid = lax.axis_index("c")
    sid = lax.axis_index("s")
    wid = cid * NS + sid
    pltpu.sync_copy(zer.at[pl.ds(sid * RPS, RPS)], acc.at[pl.ds(sid * RPS, RPS)])
    pltpu.sync_copy(srcw.at[pl.ds(wid * CHD, CHD)], sidx)
    pltpu.sync_copy(dstw.at[pl.ds(wid * CHD, CHD)], didx)
    plsc.subcore_barrier()

    def body(j, c):
        pltpu.async_copy(hp.at[sidx.at[j]], rows, sem).wait()
        pltpu.sync_copy(rows, acc.at[didx.at[j]], add=True)
        return c

    lax.fori_loop(0, CHD, body, 0)
    plsc.subcore_barrier()
    pltpu.sync_copy(acc.at[pl.ds(sid * RPS, RPS)], out.at[cid, pl.ds(sid * RPS, RPS)])


def _dinv_body(d0, d1, o):
    o[...] = lax.rsqrt(d0[...] + d1[...] + 1.0)


def _dinv(d0, d1):
    return pl.pallas_call(
        _dinv_body,
        out_shape=jax.ShapeDtypeStruct(d0.shape, jnp.float32),
    )(d0, d1)


def _mm_scale_body(h, w, dv, o):
    o[...] = (
        jnp.dot(h[...], w[...], preferred_element_type=jnp.float32,
                precision=lax.Precision.HIGHEST)
        * dv[...]
    )


def _mm_scale(h, w, dinv_col):
    return pl.pallas_call(
        _mm_scale_body,
        grid=(GB,),
        in_specs=[
            pl.BlockSpec((ROWBLK, D), lambda i: (i, 0)),
            pl.BlockSpec((D, D), lambda i: (0, 0)),
            pl.BlockSpec((ROWBLK, 1), lambda i: (i, 0)),
        ],
        out_specs=pl.BlockSpec((ROWBLK, D), lambda i: (i, 0)),
        out_shape=jax.ShapeDtypeStruct((N, D), jnp.float32),
    )(h, w, dinv_col)


def _finish_body(a0, a1, hp, dv, b, g, be, res, o):
    t = (a0[...] + a1[...] + hp[...]) * dv[...] + b[...]
    mu = jnp.mean(t, axis=-1, keepdims=True)
    xc = t - mu
    var = jnp.mean(xc * xc, axis=-1, keepdims=True)
    y = xc * lax.rsqrt(var + 1e-5) * g[...] + be[...]
    e = jnp.where(y > 0, y, jnp.exp(y) - 1.0)
    o[...] = e + res[...]


def _finish(a0, a1, hp, dinv_col, b, g, be, res):
    blk = lambda i: (i, 0)
    return pl.pallas_call(
        _finish_body,
        grid=(GB,),
        in_specs=[
            pl.BlockSpec((ROWBLK, D), blk),
            pl.BlockSpec((ROWBLK, D), blk),
            pl.BlockSpec((ROWBLK, D), blk),
            pl.BlockSpec((ROWBLK, 1), blk),
            pl.BlockSpec((1, D), lambda i: (0, 0)),
            pl.BlockSpec((1, D), lambda i: (0, 0)),
            pl.BlockSpec((1, D), lambda i: (0, 0)),
            pl.BlockSpec((ROWBLK, D), blk),
        ],
        out_specs=pl.BlockSpec((ROWBLK, D), blk),
        out_shape=jax.ShapeDtypeStruct((N, D), jnp.float32),
    )(a0, a1, hp, dinv_col, b, g, be, res)


def _head_body(h, w1, b1, g, be, w2, b2, o):
    m = (
        jnp.dot(h[...], w1[...], preferred_element_type=jnp.float32,
                precision=lax.Precision.HIGHEST)
        + b1[...]
    )
    mu = jnp.mean(m, axis=-1, keepdims=True)
    xc = m - mu
    var = jnp.mean(xc * xc, axis=-1, keepdims=True)
    y = xc * lax.rsqrt(var + 1e-5) * g[...] + be[...]
    e = jnp.where(y > 0, y, jnp.exp(y) - 1.0)
    lg = (
        jnp.dot(e, w2[...], preferred_element_type=jnp.float32,
                precision=lax.Precision.HIGHEST)
        + b2[...]
    )
    z = lg - jnp.max(lg, axis=-1, keepdims=True)
    ez = jnp.exp(z)
    o[...] = ez / jnp.sum(ez, axis=-1, keepdims=True)


def _head(h, w1, b1, g, be, w2, b2):
    return pl.pallas_call(
        _head_body,
        grid=(GB,),
        in_specs=[
            pl.BlockSpec((ROWBLK, D), lambda i: (i, 0)),
            pl.BlockSpec((D, H), lambda i: (0, 0)),
            pl.BlockSpec((1, H), lambda i: (0, 0)),
            pl.BlockSpec((1, H), lambda i: (0, 0)),
            pl.BlockSpec((1, H), lambda i: (0, 0)),
            pl.BlockSpec((H, K), lambda i: (0, 0)),
            pl.BlockSpec((1, K), lambda i: (0, 0)),
        ],
        out_specs=pl.BlockSpec((ROWBLK, K), lambda i: (i, 0)),
        out_shape=jax.ShapeDtypeStruct((N, K), jnp.float32),
    )(h, w1, b1, g, be, w2, b2)


def kernel(x, edge_index, W1, b1, g1, be1, W2, b2, g2, be2, Wm1, bm1, gm, bem, Wm2, bm2):
    src = edge_index[0]
    dst = edge_index[1]
    pad = EPAD - E
    ppat = jnp.arange(pad, dtype=jnp.int32)
    srcw = jnp.concatenate([src, ppat % CHUNK]).reshape(TOTCH, CHUNK)
    dstw = jnp.concatenate([dst, N + ppat % NDUMP]).reshape(TOTCH, CHUNK)
    zer2 = jnp.zeros((NP, D), jnp.float32)
    zer1 = jnp.zeros((NPD,), jnp.float32)

    degp = _sc_degree(dstw, zer1)
    dinv2d = _dinv(degp[0].reshape(NPD // 128, 128), degp[1].reshape(NPD // 128, 128))
    dinv_col = dinv2d.reshape(NPD)[:N].reshape(N, 1)

    b1r, g1r, be1r = b1.reshape(1, D), g1.reshape(1, D), be1.reshape(1, D)
    b2r, g2r, be2r = b2.reshape(1, D), g2.reshape(1, D), be2.reshape(1, D)

    h = x
    for (W, bb, gg, bebe) in ((W1, b1r, g1r, be1r), (W2, b2r, g2r, be2r)):
        hp = _mm_scale(h, W, dinv_col)
        agg = _sc_spmm(hp, srcw, dstw, zer2)
        h = _finish(agg[0, :N], agg[1, :N], hp, dinv_col, bb, gg, bebe, h)

    return _head(
        h, Wm1, bm1.reshape(1, H), gm.reshape(1, H), bem.reshape(1, H),
        Wm2, bm2.reshape(1, K),
    )
